# baseline (speedup 1.0000x reference)
import jax
import jax.numpy as jnp
from jax import lax
from jax.experimental import pallas as pl
from jax.experimental.pallas import tpu as pltpu
from jax.experimental.pallas import tpu_sc as plsc

NUM_CLASSES = 26
HIDDEN = 128
LANES = 16

_info = plsc.get_sparse_core_info()
_NC, _NS = _info.num_cores, _info.num_subcores
_NW = _NC * _NS


def _bc(vec, lane):
    idx = jnp.full((LANES,), lane, dtype=jnp.int32)
    return vec.at[idx].get(mode="promise_in_bounds", unique_indices=False)


def _sc_body(x_hbm, tab_hbm, out_hbm, tab_v, inp_v, out_v):
    bw = inp_v.shape[0]
    seq = inp_v.shape[1]
    wid = lax.axis_index("s") * _NC + lax.axis_index("c")
    base = wid * bw
    pltpu.sync_copy(tab_hbm, tab_v)
    pltpu.sync_copy(x_hbm.at[pl.ds(base, bw)], inp_v)

    inv = jnp.float32(1.0 / NUM_CLASSES)
    for blk in range(HIDDEN // (2 * LANES)):
        lo = blk * 2 * LANES
        tab_a = [tab_v[c, pl.ds(lo, LANES)] * inv for c in range(NUM_CLASSES)]
        tab_b = [tab_v[c, pl.ds(lo + LANES, LANES)] * inv
                 for c in range(NUM_CLASSES)]

        def brow(b, _, lo=lo, tab_a=tab_a, tab_b=tab_b):
            def body(t, _2):
                w0 = inp_v[b, t, pl.ds(0, LANES)]
                w1 = inp_v[b, t, pl.ds(NUM_CLASSES - LANES, LANES)]
                w = [_bc(w0, c) for c in range(LANES)]
                w += [_bc(w1, c - (NUM_CLASSES - LANES))
                      for c in range(LANES, NUM_CLASSES)]
                acc_a = w[0] * tab_a[0]
                acc_b = w[0] * tab_b[0]
                for c in range(1, NUM_CLASSES):
                    acc_a += w[c] * tab_a[c]
                    acc_b += w[c] * tab_b[c]
                out_v[b, t, pl.ds(lo, LANES)] = acc_a
                out_v[b, t, pl.ds(lo + LANES, LANES)] = acc_b
                return _2

            lax.fori_loop(0, seq, body, None, unroll=2)
            return _

        lax.fori_loop(0, bw, brow, None)

    pltpu.sync_copy(out_v, out_hbm.at[pl.ds(base, bw)])


def kernel(inputs, emb_table):
    B, L, C = inputs.shape
    bw = B // _NW
    return pl.kernel(
        _sc_body,
        out_type=jax.ShapeDtypeStruct((B, L, HIDDEN), jnp.float32),
        mesh=plsc.VectorSubcoreMesh(core_axis_name="c", subcore_axis_name="s"),
        compiler_params=pltpu.CompilerParams(use_tc_tiling_on_sc=False),
        scratch_types=[
            pltpu.VMEM((NUM_CLASSES, HIDDEN), jnp.float32),
            pltpu.VMEM((bw, L, C), jnp.float32),
            pltpu.VMEM((bw, L, HIDDEN), jnp.float32),
        ],
    )(inputs, emb_table)


# speedup vs baseline: 1.1511x; 1.1511x over previous
import jax
import jax.numpy as jnp
from jax import lax
from jax.experimental import pallas as pl
from jax.experimental.pallas import tpu as pltpu
from jax.experimental.pallas import tpu_sc as plsc

NUM_CLASSES = 26
HIDDEN = 128
LANES = 16

_info = plsc.get_sparse_core_info()
_NC, _NS = _info.num_cores, _info.num_subcores
_NW = _NC * _NS


def _bc(vec, lane):
    idx = jnp.full((LANES,), lane, dtype=jnp.int32)
    return vec.at[idx].get(mode="promise_in_bounds", unique_indices=False)


def _sc_body(x_hbm, tab_hbm, out_hbm, tab_v, inp_v, out_v):
    bw = inp_v.shape[0]
    seq = inp_v.shape[1]
    wid = lax.axis_index("s") * _NC + lax.axis_index("c")
    base = wid * bw
    pltpu.sync_copy(tab_hbm, tab_v)
    pltpu.sync_copy(x_hbm.at[pl.ds(base, bw)], inp_v)

    inv = jnp.float32(1.0 / NUM_CLASSES)
    for blk in range(HIDDEN // (2 * LANES)):
        lo = blk * 2 * LANES
        tab_a = [tab_v[c, pl.ds(lo, LANES)] * inv for c in range(NUM_CLASSES)]
        tab_b = [tab_v[c, pl.ds(lo + LANES, LANES)] * inv
                 for c in range(NUM_CLASSES)]

        def brow(b, _, lo=lo, tab_a=tab_a, tab_b=tab_b):
            def body(t, _2):
                w0 = inp_v[b, t, pl.ds(0, LANES)]
                w1 = inp_v[b, t, pl.ds(NUM_CLASSES - LANES, LANES)]
                w = [_bc(w0, c) for c in range(LANES)]
                w += [_bc(w1, c - (NUM_CLASSES - LANES))
                      for c in range(LANES, NUM_CLASSES)]
                acc_a = w[0] * tab_a[0]
                acc_b = w[0] * tab_b[0]
                for c in range(1, NUM_CLASSES):
                    acc_a += w[c] * tab_a[c]
                    acc_b += w[c] * tab_b[c]
                out_v[t, b, pl.ds(lo, LANES)] = acc_a
                out_v[t, b, pl.ds(lo + LANES, LANES)] = acc_b
                return _2

            lax.fori_loop(0, seq, body, None, unroll=2)
            return _

        lax.fori_loop(0, bw, brow, None)

    pltpu.sync_copy(out_v, out_hbm.at[:, pl.ds(base, bw)])


def kernel(inputs, emb_table):
    B, L, C = inputs.shape
    bw = B // _NW
    out_t = pl.kernel(
        _sc_body,
        out_type=jax.ShapeDtypeStruct((L, B, HIDDEN), jnp.float32),
        mesh=plsc.VectorSubcoreMesh(core_axis_name="c", subcore_axis_name="s"),
        compiler_params=pltpu.CompilerParams(use_tc_tiling_on_sc=False),
        scratch_types=[
            pltpu.VMEM((NUM_CLASSES, HIDDEN), jnp.float32),
            pltpu.VMEM((bw, L, C), jnp.float32),
            pltpu.VMEM((L, bw, HIDDEN), jnp.float32),
        ],
    )(inputs, emb_table)
    return jnp.transpose(out_t, (1, 0, 2))


# speedup vs baseline: 1.1967x; 1.0396x over previous
import jax
import jax.numpy as jnp
from jax import lax
from jax.experimental import pallas as pl
from jax.experimental.pallas import tpu as pltpu
from jax.experimental.pallas import tpu_sc as plsc

NUM_CLASSES = 26
HIDDEN = 128
LANES = 16
LPAD = 24

_info = plsc.get_sparse_core_info()
_NC, _NS = _info.num_cores, _info.num_subcores
_NW = _NC * _NS


def _sc_body(x_hbm, tab_hbm, out_hbm, tab_v, inp_v, out_v):
    bw = inp_v.shape[0]
    seq = out_v.shape[0]
    wid = lax.axis_index("s") * _NC + lax.axis_index("c")
    base = wid * bw
    pltpu.sync_copy(tab_hbm, tab_v)
    pltpu.sync_copy(x_hbm.at[pl.ds(base, bw)], inp_v)

    inv = jnp.float32(1.0 / NUM_CLASSES)
    for blk in range(HIDDEN // (2 * LANES)):
        lo = blk * 2 * LANES
        tab_a = [tab_v[c, pl.ds(lo, LANES)] * inv for c in range(NUM_CLASSES)]
        tab_b = [tab_v[c, pl.ds(lo + LANES, LANES)] * inv
                 for c in range(NUM_CLASSES)]

        def brow(b, _, lo=lo, tab_a=tab_a, tab_b=tab_b):
            def body(t, _2):
                w0 = inp_v[b, t, pl.ds(0, LANES)]
                w1 = inp_v[b, t, pl.ds(NUM_CLASSES - LANES, LANES)]
                w = [w0[c] for c in range(LANES)]
                w += [w1[c - (NUM_CLASSES - LANES)]
                      for c in range(LANES, NUM_CLASSES)]
                acc_a = w[0] * tab_a[0]
                acc_b = w[0] * tab_b[0]
                for c in range(1, NUM_CLASSES):
                    acc_a += w[c] * tab_a[c]
                    acc_b += w[c] * tab_b[c]
                out_v[t, b, pl.ds(lo, LANES)] = acc_a
                out_v[t, b, pl.ds(lo + LANES, LANES)] = acc_b
                return _2

            lax.fori_loop(0, seq, body, None, unroll=2)
            return _

        lax.fori_loop(0, bw, brow, None)

    pltpu.sync_copy(out_v, out_hbm.at[:, pl.ds(base, bw)])


def kernel(inputs, emb_table):
    B, L, C = inputs.shape
    bw = B // _NW
    xp = jnp.pad(inputs, ((0, 0), (0, LPAD - L), (0, 0)))
    out_t = pl.kernel(
        _sc_body,
        out_type=jax.ShapeDtypeStruct((L, B, HIDDEN), jnp.float32),
        mesh=plsc.VectorSubcoreMesh(core_axis_name="c", subcore_axis_name="s"),
        compiler_params=pltpu.CompilerParams(use_tc_tiling_on_sc=False),
        scratch_types=[
            pltpu.VMEM((NUM_CLASSES, HIDDEN), jnp.float32),
            pltpu.VMEM((bw, LPAD, C), jnp.float32),
            pltpu.VMEM((L, bw, HIDDEN), jnp.float32),
        ],
    )(xp, emb_table)
    return jnp.transpose(out_t, (1, 0, 2))


# speedup vs baseline: 1.3927x; 1.1638x over previous
import jax
import jax.numpy as jnp
from jax import lax
from jax.experimental import pallas as pl
from jax.experimental.pallas import tpu as pltpu
from jax.experimental.pallas import tpu_sc as plsc

NUM_CLASSES = 26
HIDDEN = 128
LANES = 16
LPAD = 24
CPAD = 32

_info = plsc.get_sparse_core_info()
_NC, _NS = _info.num_cores, _info.num_subcores
_NW = _NC * _NS


def _sc_body(x_hbm, tab_hbm, out_hbm, tab_v, inp_v, out_v):
    bw = inp_v.shape[0]
    seq = out_v.shape[0]
    wid = lax.axis_index("s") * _NC + lax.axis_index("c")
    base = wid * bw
    pltpu.sync_copy(tab_hbm, tab_v)
    pltpu.sync_copy(x_hbm.at[pl.ds(base, bw)], inp_v)

    inv = jnp.float32(1.0 / NUM_CLASSES)
    for blk in range(HIDDEN // (2 * LANES)):
        lo = blk * 2 * LANES
        tab_a = [tab_v[c, pl.ds(lo, LANES)] * inv for c in range(NUM_CLASSES)]
        tab_b = [tab_v[c, pl.ds(lo + LANES, LANES)] * inv
                 for c in range(NUM_CLASSES)]

        def brow(b, _, lo=lo, tab_a=tab_a, tab_b=tab_b):
            @plsc.parallel_loop(0, seq, unroll=2)
            def body(t):
                w0 = inp_v[b, t, pl.ds(0, LANES)]
                w1 = inp_v[b, t, pl.ds(NUM_CLASSES - LANES, LANES)]
                w = [w0[c] for c in range(LANES)]
                w += [w1[c - (NUM_CLASSES - LANES)]
                      for c in range(LANES, NUM_CLASSES)]
                acc_a = w[0] * tab_a[0]
                acc_b = w[0] * tab_b[0]
                for c in range(1, NUM_CLASSES):
                    acc_a += w[c] * tab_a[c]
                    acc_b += w[c] * tab_b[c]
                out_v[t, b, pl.ds(lo, LANES)] = acc_a
                out_v[t, b, pl.ds(lo + LANES, LANES)] = acc_b

            return _

        lax.fori_loop(0, bw, brow, None)

    pltpu.sync_copy(out_v, out_hbm.at[:, pl.ds(base, bw)])


def kernel(inputs, emb_table):
    B, L, C = inputs.shape
    bw = B // _NW
    xp = jnp.pad(inputs, ((0, 0), (0, LPAD - L), (0, CPAD - C)))
    out_t = pl.kernel(
        _sc_body,
        out_type=jax.ShapeDtypeStruct((L, B, HIDDEN), jnp.float32),
        mesh=plsc.VectorSubcoreMesh(core_axis_name="c", subcore_axis_name="s"),
        compiler_params=pltpu.CompilerParams(use_tc_tiling_on_sc=False),
        scratch_types=[
            pltpu.VMEM((NUM_CLASSES, HIDDEN), jnp.float32),
            pltpu.VMEM((bw, LPAD, CPAD), jnp.float32),
            pltpu.VMEM((L, bw, HIDDEN), jnp.float32),
        ],
    )(xp, emb_table)
    return jnp.transpose(out_t, (1, 0, 2))


# speedup vs baseline: 1.5480x; 1.1115x over previous
import jax
import jax.numpy as jnp
from jax import lax
from jax.experimental import pallas as pl
from jax.experimental.pallas import tpu as pltpu
from jax.experimental.pallas import tpu_sc as plsc

NUM_CLASSES = 26
HIDDEN = 128
LANES = 16
LPAD = 24
CPAD = 32

_info = plsc.get_sparse_core_info()
_NC, _NS = _info.num_cores, _info.num_subcores
_NW = _NC * _NS


def _sc_body(x_hbm, tab_hbm, out_hbm, tab_v, inp_v, out_v):
    bw = inp_v.shape[0]
    seq = out_v.shape[0]
    wid = lax.axis_index("s") * _NC + lax.axis_index("c")
    base = wid * bw
    pltpu.sync_copy(tab_hbm, tab_v)
    pltpu.sync_copy(x_hbm.at[pl.ds(base, bw), :, pl.ds(0, CPAD)], inp_v)

    inv = jnp.float32(1.0 / NUM_CLASSES)
    for blk in range(HIDDEN // (2 * LANES)):
        lo = blk * 2 * LANES
        tab_a = [tab_v[c, pl.ds(lo, LANES)] * inv for c in range(NUM_CLASSES)]
        tab_b = [tab_v[c, pl.ds(lo + LANES, LANES)] * inv
                 for c in range(NUM_CLASSES)]

        def brow(b, _, lo=lo, tab_a=tab_a, tab_b=tab_b):
            @plsc.parallel_loop(0, seq, unroll=4)
            def body(t):
                w0 = inp_v[b, t, pl.ds(0, LANES)]
                w1 = inp_v[b, t, pl.ds(NUM_CLASSES - LANES, LANES)]
                w = [w0[c] for c in range(LANES)]
                w += [w1[c - (NUM_CLASSES - LANES)]
                      for c in range(LANES, NUM_CLASSES)]
                acc_a = w[0] * tab_a[0]
                acc_b = w[0] * tab_b[0]
                for c in range(1, NUM_CLASSES):
                    acc_a += w[c] * tab_a[c]
                    acc_b += w[c] * tab_b[c]
                out_v[t, b, pl.ds(lo, LANES)] = acc_a
                out_v[t, b, pl.ds(lo + LANES, LANES)] = acc_b

            return _

        lax.fori_loop(0, bw, brow, None)

    pltpu.sync_copy(out_v, out_hbm.at[:, pl.ds(base, bw)])


def kernel(inputs, emb_table):
    B, L, C = inputs.shape
    bw = B // _NW
    xp = jnp.pad(inputs, ((0, 0), (0, LPAD - L), (0, HIDDEN - C)))
    out_t = pl.kernel(
        _sc_body,
        out_type=jax.ShapeDtypeStruct((L, B, HIDDEN), jnp.float32),
        mesh=plsc.VectorSubcoreMesh(core_axis_name="c", subcore_axis_name="s"),
        compiler_params=pltpu.CompilerParams(use_tc_tiling_on_sc=False),
        scratch_types=[
            pltpu.VMEM((NUM_CLASSES, HIDDEN), jnp.float32),
            pltpu.VMEM((bw, LPAD, CPAD), jnp.float32),
            pltpu.VMEM((L, bw, HIDDEN), jnp.float32),
        ],
    )(xp, emb_table)
    return jnp.transpose(out_t, (1, 0, 2))
